# baseline (speedup 1.0000x reference)
import functools
import jax
import jax.numpy as jnp
from jax import lax
from jax.experimental import pallas as pl
from jax.experimental.pallas import tpu as pltpu, tpu_sc as plsc

G = 16


def _sc_gather_t(idx2, Wr, D):
    NW, rpw = idx2.shape
    L = 16
    ngrp = rpw // L
    info = plsc.get_sparse_core_info()
    nc = info.num_cores
    mesh = plsc.VectorSubcoreMesh(core_axis_name="c", subcore_axis_name="s")

    NQ = 2
    qw = rpw // NQ

    @functools.partial(
        pl.kernel,
        mesh=mesh,
        out_type=jax.ShapeDtypeStruct((NW, NQ, D, qw), jnp.float32),
        scratch_types=[
            pltpu.VMEM((rpw,), jnp.int32),
            pltpu.VMEM(Wr.shape, jnp.float32),
            pltpu.VMEM((D, rpw), jnp.float32),
        ],
        compiler_params=pltpu.CompilerParams(needs_layout_passes=False),
    )
    def k(idx_hbm, table_hbm, out_hbm, idx_v, tab_v, rows_v):
        wid = lax.axis_index("s") * nc + lax.axis_index("c")
        pltpu.sync_copy(idx_hbm.at[wid], idx_v)
        pltpu.sync_copy(table_hbm, tab_v)

        def grp(g, _):
            off = pl.multiple_of(g * L, L)
            tok = idx_v[pl.ds(off, L)]
            base = tok * D
            for c in range(D):
                vals = plsc.load_gather(tab_v, [base + c])
                rows_v[c, pl.ds(off, L)] = vals
            return 0

        lax.fori_loop(0, ngrp, grp, 0)
        for q in range(NQ):
            pltpu.sync_copy(rows_v.at[:, pl.ds(q * qw, qw)],
                            out_hbm.at[wid, q])

    return k(idx2, Wr)


def _tc_body(wc_ref, wr_ref, st_ref, en_ref, we_ref, b_ref, o_ref):
    T = wc_ref.shape[1]
    S = st_ref.shape[1]
    D = b_ref.shape[1]
    rr = jax.lax.broadcasted_iota(jnp.int32, (T, T), 0)
    cc = jax.lax.broadcasted_iota(jnp.int32, (T, T), 1)
    pos = jax.lax.broadcasted_iota(jnp.int32, (S, T), 1)
    for g in range(G):
        wc = wc_ref[0][:, g:g + 1]
        wr = wr_ref[0][g:g + 1, :]
        eq = (wc == wr) & (rr < cc)
        prev2 = jnp.max(jnp.where(eq, rr, -1), axis=0,
                        keepdims=True)
        st = st_ref[0][:, g:g + 1]
        en = en_ref[0][:, g:g + 1]
        m = (pos >= st) & (pos < en) & (prev2 < st)
        wet = we_ref[0, 0][:, g * T:(g + 1) * T]
        o_ref[0, :, g * D:(g + 1) * D] = lax.dot_general(
            m.astype(jnp.float32), wet,
            dimension_numbers=(((1,), (1,)), ((), ())),
            preferred_element_type=jnp.float32) + b_ref[...]


def kernel(word_encs, span_idxs, W, bias):
    B, T = word_encs.shape
    S = span_idxs.shape[1]
    V, D = W.shape
    NW = 32
    w32 = word_encs.astype(jnp.int32)
    Wf = W.astype(jnp.float32)
    WEt = _sc_gather_t(w32.reshape(NW, B * T // NW),
                       Wf.reshape(V * D), D)
    NB = B // G
    wr3 = w32.reshape(NB, G, T)
    wc3 = jnp.transpose(wr3, (0, 2, 1))
    si3 = jnp.transpose(span_idxs.astype(jnp.int32).reshape(NB, G, S, 2),
                        (0, 2, 1, 3))
    st3 = si3[:, :, :, 0]
    en3 = si3[:, :, :, 1]
    bias2 = bias.astype(jnp.float32).reshape(1, D)
    bpw = B // NW
    nblk = bpw // G
    out3 = pl.pallas_call(
        _tc_body,
        grid=(NB,),
        in_specs=[
            pl.BlockSpec((1, T, G), lambda i: (i, 0, 0)),
            pl.BlockSpec((1, G, T), lambda i: (i, 0, 0)),
            pl.BlockSpec((1, S, G), lambda i: (i, 0, 0)),
            pl.BlockSpec((1, S, G), lambda i: (i, 0, 0)),
            pl.BlockSpec((1, 1, D, G * T),
                         lambda i: (i // nblk, i % nblk, 0, 0)),
            pl.BlockSpec((1, D), lambda i: (0, 0)),
        ],
        out_specs=pl.BlockSpec((1, S, G * D), lambda i: (i, 0, 0)),
        out_shape=jax.ShapeDtypeStruct((NB, S, G * D), jnp.float32),
    )(wc3, wr3, st3, en3, WEt, bias2)
    out = jnp.transpose(out3.reshape(NB, S, G, D), (0, 2, 1, 3))
    return out.reshape(B, S, D)


# speedup vs baseline: 1.5872x; 1.5872x over previous
import functools
import jax
import jax.numpy as jnp
from jax import lax
from jax.experimental import pallas as pl
from jax.experimental.pallas import tpu as pltpu, tpu_sc as plsc

G = 16


def _sc_gather_t(idx2, Wr, D):
    NW, rpw = idx2.shape
    L = 16
    ngrp = rpw // L
    info = plsc.get_sparse_core_info()
    nc = info.num_cores
    mesh = plsc.VectorSubcoreMesh(core_axis_name="c", subcore_axis_name="s")

    NQ = 2
    qw = rpw // NQ

    @functools.partial(
        pl.kernel,
        mesh=mesh,
        out_type=jax.ShapeDtypeStruct((NW, NQ, D, qw), jnp.float32),
        scratch_types=[
            pltpu.VMEM((rpw,), jnp.int32),
            pltpu.VMEM(Wr.shape, jnp.float32),
            pltpu.VMEM((D, rpw), jnp.float32),
        ],
        compiler_params=pltpu.CompilerParams(needs_layout_passes=False),
    )
    def k(idx_hbm, table_hbm, out_hbm, idx_v, tab_v, rows_v):
        wid = lax.axis_index("s") * nc + lax.axis_index("c")
        pltpu.sync_copy(idx_hbm.at[wid], idx_v)
        pltpu.sync_copy(table_hbm, tab_v)

        def grp(g, _):
            off = pl.multiple_of(g * L, L)
            tok = idx_v[pl.ds(off, L)]
            base = tok * D
            for c in range(D):
                vals = plsc.load_gather(tab_v, [base + c])
                rows_v[c, pl.ds(off, L)] = vals
            return 0

        lax.fori_loop(0, ngrp, grp, 0)
        for q in range(NQ):
            pltpu.sync_copy(rows_v.at[:, pl.ds(q * qw, qw)],
                            out_hbm.at[wid, q])

    return k(idx2, Wr)


def _tc_body(wc_ref, wr_ref, st_ref, en_ref, we_ref, b_ref, o_ref):
    T = wc_ref.shape[1]
    S = st_ref.shape[1]
    rr = jax.lax.broadcasted_iota(jnp.int32, (T, T), 0)
    cc = jax.lax.broadcasted_iota(jnp.int32, (T, T), 1)
    pos = jax.lax.broadcasted_iota(jnp.int32, (S, T), 1)
    for g in range(G):
        wc = wc_ref[0][:, g:g + 1]
        wr = wr_ref[0][g:g + 1, :]
        eq = (wc == wr) & (rr < cc)
        prev2 = jnp.max(jnp.where(eq, rr, -1), axis=0,
                        keepdims=True)
        st = st_ref[0][:, g:g + 1]
        en = en_ref[0][:, g:g + 1]
        m = (pos >= st) & (pos < en) & (prev2 < st)
        wet = we_ref[0, 0][:, g * T:(g + 1) * T]
        o_ref[g] = lax.dot_general(
            m.astype(jnp.float32), wet,
            dimension_numbers=(((1,), (1,)), ((), ())),
            preferred_element_type=jnp.float32) + b_ref[...]


def kernel(word_encs, span_idxs, W, bias):
    B, T = word_encs.shape
    S = span_idxs.shape[1]
    V, D = W.shape
    NW = 32
    w32 = word_encs.astype(jnp.int32)
    Wf = W.astype(jnp.float32)
    WEt = _sc_gather_t(w32.reshape(NW, B * T // NW),
                       Wf.reshape(V * D), D)
    NB = B // G
    wr3 = w32.reshape(NB, G, T)
    wc3 = jnp.transpose(wr3, (0, 2, 1))
    si3 = jnp.transpose(span_idxs.astype(jnp.int32).reshape(NB, G, S, 2),
                        (0, 2, 1, 3))
    st3 = si3[:, :, :, 0]
    en3 = si3[:, :, :, 1]
    bias2 = bias.astype(jnp.float32).reshape(1, D)
    bpw = B // NW
    nblk = bpw // G
    out = pl.pallas_call(
        _tc_body,
        grid=(NB,),
        in_specs=[
            pl.BlockSpec((1, T, G), lambda i: (i, 0, 0)),
            pl.BlockSpec((1, G, T), lambda i: (i, 0, 0)),
            pl.BlockSpec((1, S, G), lambda i: (i, 0, 0)),
            pl.BlockSpec((1, S, G), lambda i: (i, 0, 0)),
            pl.BlockSpec((1, 1, D, G * T),
                         lambda i: (i // nblk, i % nblk, 0, 0)),
            pl.BlockSpec((1, D), lambda i: (0, 0)),
        ],
        out_specs=pl.BlockSpec((G, S, D), lambda i: (i, 0, 0)),
        out_shape=jax.ShapeDtypeStruct((B, S, D), jnp.float32),
    )(wc3, wr3, st3, en3, WEt, bias2)
    return out


# speedup vs baseline: 1.9273x; 1.2143x over previous
import functools
import jax
import jax.numpy as jnp
from jax import lax
from jax.experimental import pallas as pl
from jax.experimental.pallas import tpu as pltpu, tpu_sc as plsc

G = 16


def _sc_gather_t(idx2, Wr, D):
    NW, rpw = idx2.shape
    L = 16
    ngrp = rpw // L
    info = plsc.get_sparse_core_info()
    nc = info.num_cores
    mesh = plsc.VectorSubcoreMesh(core_axis_name="c", subcore_axis_name="s")

    NQ = 2
    qw = rpw // NQ

    @functools.partial(
        pl.kernel,
        mesh=mesh,
        out_type=jax.ShapeDtypeStruct((NW, NQ, D, qw), jnp.float32),
        scratch_types=[
            pltpu.VMEM((rpw,), jnp.int32),
            pltpu.VMEM(Wr.shape, jnp.float32),
            pltpu.VMEM((D, rpw), jnp.float32),
        ],
        compiler_params=pltpu.CompilerParams(needs_layout_passes=False),
    )
    def k(idx_hbm, table_hbm, out_hbm, idx_v, tab_v, rows_v):
        wid = lax.axis_index("s") * nc + lax.axis_index("c")
        pltpu.sync_copy(idx_hbm.at[wid], idx_v)
        pltpu.sync_copy(table_hbm, tab_v)

        @plsc.parallel_loop(0, ngrp, 1, unroll=4)
        def grp(g):
            off = pl.multiple_of(g * L, L)
            tok = idx_v[pl.ds(off, L)]
            base = tok * D
            for c in range(D):
                vals = plsc.load_gather(tab_v, [base + c])
                rows_v[c, pl.ds(off, L)] = vals
        for q in range(NQ):
            pltpu.sync_copy(rows_v.at[:, pl.ds(q * qw, qw)],
                            out_hbm.at[wid, q])

    return k(idx2, Wr)


def _tc_body(wc_ref, wr_ref, st_ref, en_ref, we_ref, b_ref, o_ref):
    T = wc_ref.shape[1]
    S = st_ref.shape[1]
    rr = jax.lax.broadcasted_iota(jnp.int32, (T, T), 0)
    cc = jax.lax.broadcasted_iota(jnp.int32, (T, T), 1)
    pos = jax.lax.broadcasted_iota(jnp.int32, (S, T), 1)
    for g in range(G):
        wc = wc_ref[0][:, g:g + 1]
        wr = wr_ref[0][g:g + 1, :]
        eq = (wc == wr) & (rr < cc)
        prev2 = jnp.max(jnp.where(eq, rr, -1), axis=0,
                        keepdims=True)
        st = st_ref[0][:, g:g + 1]
        en = en_ref[0][:, g:g + 1]
        m = (pos >= st) & (pos < en) & (prev2 < st)
        wet = we_ref[0, 0][:, g * T:(g + 1) * T]
        o_ref[g] = lax.dot_general(
            m.astype(jnp.float32), wet,
            dimension_numbers=(((1,), (1,)), ((), ())),
            preferred_element_type=jnp.float32) + b_ref[...]


def kernel(word_encs, span_idxs, W, bias):
    B, T = word_encs.shape
    S = span_idxs.shape[1]
    V, D = W.shape
    NW = 32
    w32 = word_encs.astype(jnp.int32)
    Wf = W.astype(jnp.float32)
    WEt = _sc_gather_t(w32.reshape(NW, B * T // NW),
                       Wf.reshape(V * D), D)
    NB = B // G
    wr3 = w32.reshape(NB, G, T)
    wc3 = jnp.transpose(wr3, (0, 2, 1))
    si3 = jnp.transpose(span_idxs.astype(jnp.int32).reshape(NB, G, S, 2),
                        (0, 2, 1, 3))
    st3 = si3[:, :, :, 0]
    en3 = si3[:, :, :, 1]
    bias2 = bias.astype(jnp.float32).reshape(1, D)
    bpw = B // NW
    nblk = bpw // G
    out = pl.pallas_call(
        _tc_body,
        grid=(NB,),
        in_specs=[
            pl.BlockSpec((1, T, G), lambda i: (i, 0, 0)),
            pl.BlockSpec((1, G, T), lambda i: (i, 0, 0)),
            pl.BlockSpec((1, S, G), lambda i: (i, 0, 0)),
            pl.BlockSpec((1, S, G), lambda i: (i, 0, 0)),
            pl.BlockSpec((1, 1, D, G * T),
                         lambda i: (i // nblk, i % nblk, 0, 0)),
            pl.BlockSpec((1, D), lambda i: (0, 0)),
        ],
        out_specs=pl.BlockSpec((G, S, D), lambda i: (i, 0, 0)),
        out_shape=jax.ShapeDtypeStruct((B, S, D), jnp.float32),
    )(wc3, wr3, st3, en3, WEt, bias2)
    return out


# speedup vs baseline: 2.0194x; 1.0478x over previous
import functools
import jax
import jax.numpy as jnp
from jax import lax
from jax.experimental import pallas as pl
from jax.experimental.pallas import tpu as pltpu, tpu_sc as plsc

G = 16


def _sc_gather_t(idx2, Wr, D, NQ):
    NW, rpw = idx2.shape
    L = 16
    ngrp = rpw // L
    info = plsc.get_sparse_core_info()
    nc = info.num_cores
    mesh = plsc.VectorSubcoreMesh(core_axis_name="c", subcore_axis_name="s")

    qw = rpw // NQ

    @functools.partial(
        pl.kernel,
        mesh=mesh,
        out_type=jax.ShapeDtypeStruct((NW, NQ, D, qw), jnp.float32),
        scratch_types=[
            pltpu.VMEM((rpw,), jnp.int32),
            pltpu.VMEM(Wr.shape, jnp.float32),
            pltpu.VMEM((D, rpw), jnp.float32),
        ],
        compiler_params=pltpu.CompilerParams(needs_layout_passes=False),
    )
    def k(idx_hbm, table_hbm, out_hbm, idx_v, tab_v, rows_v):
        wid = lax.axis_index("s") * nc + lax.axis_index("c")
        pltpu.sync_copy(idx_hbm.at[wid], idx_v)
        pltpu.sync_copy(table_hbm, tab_v)

        @plsc.parallel_loop(0, ngrp, 1, unroll=4)
        def grp(g):
            off = pl.multiple_of(g * L, L)
            tok = idx_v[pl.ds(off, L)]
            base = tok * D
            for c in range(D):
                vals = plsc.load_gather(tab_v, [base + c])
                rows_v[c, pl.ds(off, L)] = vals
        for q in range(NQ):
            pltpu.sync_copy(rows_v.at[:, pl.ds(q * qw, qw)],
                            out_hbm.at[wid, q])

    return k(idx2, Wr)


def _tc_body(wc_ref, wr_ref, st_ref, en_ref, we_ref, b_ref, o_ref):
    T = wc_ref.shape[1]
    S = st_ref.shape[1]
    rr = jax.lax.broadcasted_iota(jnp.int32, (T, T), 0)
    cc = jax.lax.broadcasted_iota(jnp.int32, (T, T), 1)
    pos = jax.lax.broadcasted_iota(jnp.int32, (S, T), 1)
    for g in range(G):
        wc = wc_ref[0][:, g:g + 1]
        wr = wr_ref[0][g:g + 1, :]
        eq = (wc == wr) & (rr < cc)
        prev2 = jnp.max(jnp.where(eq, rr, -1), axis=0,
                        keepdims=True)
        st = st_ref[0][:, g:g + 1]
        en = en_ref[0][:, g:g + 1]
        m = (pos >= st) & (pos < en) & (prev2 < st)
        wet = we_ref[0, 0][:, g * T:(g + 1) * T]
        o_ref[g] = lax.dot_general(
            m.astype(jnp.float32), wet,
            dimension_numbers=(((1,), (1,)), ((), ())),
            preferred_element_type=jnp.float32) + b_ref[...]


def _chunk(w32, span_i32, Wf_flat, bias2, NW):
    Bc, T = w32.shape
    S = span_i32.shape[1]
    D = bias2.shape[1]
    bpw = Bc // NW
    nblk = bpw // G
    WEt = _sc_gather_t(w32.reshape(NW, Bc * T // NW), Wf_flat, D, nblk)
    NB = Bc // G
    wr3 = w32.reshape(NB, G, T)
    wc3 = jnp.transpose(wr3, (0, 2, 1))
    si3 = jnp.transpose(span_i32.reshape(NB, G, S, 2),
                        (0, 2, 1, 3))
    st3 = si3[:, :, :, 0]
    en3 = si3[:, :, :, 1]
    return pl.pallas_call(
        _tc_body,
        grid=(NB,),
        in_specs=[
            pl.BlockSpec((1, T, G), lambda i: (i, 0, 0)),
            pl.BlockSpec((1, G, T), lambda i: (i, 0, 0)),
            pl.BlockSpec((1, S, G), lambda i: (i, 0, 0)),
            pl.BlockSpec((1, S, G), lambda i: (i, 0, 0)),
            pl.BlockSpec((1, 1, D, G * T),
                         lambda i: (i // nblk, i % nblk, 0, 0)),
            pl.BlockSpec((1, D), lambda i: (0, 0)),
        ],
        out_specs=pl.BlockSpec((G, S, D), lambda i: (i, 0, 0)),
        out_shape=jax.ShapeDtypeStruct((Bc, S, D), jnp.float32),
    )(wc3, wr3, st3, en3, WEt, bias2)


def kernel(word_encs, span_idxs, W, bias):
    B, T = word_encs.shape
    S = span_idxs.shape[1]
    V, D = W.shape
    NW = 32
    NH = 2
    Bh = B // NH
    w32 = word_encs.astype(jnp.int32)
    sp32 = span_idxs.astype(jnp.int32)
    Wf_flat = W.astype(jnp.float32).reshape(V * D)
    bias2 = bias.astype(jnp.float32).reshape(1, D)
    outs = [
        _chunk(w32[h * Bh:(h + 1) * Bh], sp32[h * Bh:(h + 1) * Bh],
               Wf_flat, bias2, NW)
        for h in range(NH)
    ]
    return jnp.concatenate(outs, axis=0)
